# baseline (speedup 1.0000x reference)
import functools

import jax
import jax.numpy as jnp
import numpy as np
from jax import lax
from jax.experimental import pallas as pl
from jax.experimental.pallas import tpu as pltpu
from jax.experimental.pallas import tpu_sc as plsc

N = 10000
E = 320000
D = 128
HOP = 4
ALPHA = 0.15
C = 47

Np = 10240
CB = 128
CH_REAL = E // CB
CH = 2560
NSUB = 16
NW = 2 * NSUB
CPS = CH // NW
RPW = Np // NSUB

_mesh = plsc.VectorSubcoreMesh(core_axis_name="c", subcore_axis_name="s")
_f32 = jnp.float32
_i32 = jnp.int32


def _sds(shape, dtype):
    return jax.ShapeDtypeStruct(shape, dtype)


_QK_PERM = np.array(
    [(l * 16 + 2 * t) if l < 8 else ((15 - l) * 16 + 2 * t + 1)
     for t in range(8) for l in range(16)], dtype=np.int32)


@functools.partial(
    pl.kernel,
    out_type=(_sds((CH, CB, 16), _f32), _sds((NW, 1, 16), _f32)),
    mesh=_mesh,
    scratch_types=[
        pltpu.VMEM((CB,), _i32),
        pltpu.VMEM((CB,), _i32),
        pltpu.VMEM((CB, D), _f32),
        pltpu.VMEM((CB, D), _f32),
        pltpu.VMEM((CB, 16), _f32),
        pltpu.VMEM((1, 16), _f32),
    ],
)
def _sc_logits(qh, kh, dstr, srcr, e_out, wmax_out, idq, idk, qd, ks, e_buf, wm_v):
    c = lax.axis_index("c")
    s = lax.axis_index("s")
    wid = c * NSUB + s
    wm_v[0, :] = jnp.full((16,), -3e38, _f32)

    def chunk_body(j, _):
        jj = wid * CPS + j

        @pl.when(jj < CH_REAL)
        def _():
            pltpu.sync_copy(dstr.at[jj], idq)
            pltpu.sync_copy(srcr.at[jj], idk)
            pltpu.sync_copy(qh.at[idq], qd)
            pltpu.sync_copy(kh.at[idk], ks)

            @pl.loop(0, CB, unroll=4)
            def _(i):
                acc = qd[i, pl.ds(0, 16)] * ks[i, pl.ds(0, 16)]
                for t in range(1, 8):
                    acc = acc + qd[i, pl.ds(16 * t, 16)] * ks[i, pl.ds(16 * t, 16)]
                ef = acc + lax.rev(acc, (0,))
                ef = jnp.maximum(ef * 0.25, ef * 0.05)
                e_buf[i, pl.ds(0, 16)] = ef
                wm_v[0, :] = jnp.maximum(wm_v[0, :], ef)
            pltpu.sync_copy(e_buf, e_out.at[jj])
        return 0

    lax.fori_loop(0, CPS, chunk_body, 0)
    pltpu.sync_copy(wm_v, wmax_out.at[wid])


@functools.partial(
    pl.kernel,
    out_type=(_sds((CH, CB, 16), _f32), _sds((2, Np, D), _f32)),
    mesh=_mesh,
    scratch_types=[
        pltpu.VMEM((CB, 16), _f32),
        pltpu.VMEM((CB, D), _f32),
        pltpu.VMEM((CB,), _i32),
        pltpu.VMEM((NW, 1, 16), _f32),
        pltpu.VMEM((1, 16), _f32),
        pltpu.VMEM_SHARED((Np, D), _f32),
    ],
)
def _sc_softmax(e_in, dstr, wmax_in, zeros128, p_out, den_out,
                eb, pwide, dstv, wm_all, gs_v, den_sh):
    c = lax.axis_index("c")
    s = lax.axis_index("s")
    wid = c * NSUB + s

    pltpu.sync_copy(zeros128, den_sh.at[pl.ds(s * RPW, RPW)])
    pltpu.sync_copy(zeros128.at[pl.ds(0, CB)], pwide)

    pltpu.sync_copy(wmax_in, wm_all)
    gs_v[0, :] = jnp.full((16,), -3e38, _f32)

    def mx(t, _):
        gs_v[0, :] = jnp.maximum(gs_v[0, :], wm_all[t, 0, :])
        return 0
    lax.fori_loop(0, NW, mx, 0)
    gs = gs_v[0, :]
    plsc.subcore_barrier()

    def chunk_body(j, _):
        jj = wid * CPS + j

        @pl.when(jj < CH_REAL)
        def _():
            pltpu.sync_copy(e_in.at[jj], eb)
            pltpu.sync_copy(dstr.at[jj], dstv)

            @pl.loop(0, CB, unroll=4)
            def _(i):
                p = jnp.exp(eb[i, pl.ds(0, 16)] - gs)
                eb[i, pl.ds(0, 16)] = p
                pwide[i, pl.ds(0, 16)] = p
            pltpu.sync_copy(eb, p_out.at[jj])
            pltpu.sync_copy(pwide, den_sh.at[dstv], add=True)
        return 0

    lax.fori_loop(0, CPS, chunk_body, 0)
    plsc.subcore_barrier()
    pltpu.sync_copy(den_sh.at[pl.ds(s * RPW, RPW)],
                    den_out.at[c, pl.ds(s * RPW, RPW)])


@functools.partial(
    pl.kernel,
    out_type=_sds((2, Np, D), _f32),
    mesh=_mesh,
    scratch_types=[
        pltpu.VMEM((CB,), _i32),
        pltpu.VMEM((CB,), _i32),
        pltpu.VMEM((CB, 16), _f32),
        pltpu.VMEM((CB, D), _f32),
        pltpu.VMEM_SHARED((Np, D), _f32),
    ],
)
def _sc_hop(hN, p_in, srcr, dstr, zeros128, agg_out,
            srcv, dstv, p_v, hrows, agg_sh):
    c = lax.axis_index("c")
    s = lax.axis_index("s")
    wid = c * NSUB + s

    pltpu.sync_copy(zeros128, agg_sh.at[pl.ds(s * RPW, RPW)])
    plsc.subcore_barrier()

    def chunk_body(j, _):
        jj = wid * CPS + j

        @pl.when(jj < CH_REAL)
        def _():
            pltpu.sync_copy(srcr.at[jj], srcv)
            pltpu.sync_copy(dstr.at[jj], dstv)
            pltpu.sync_copy(p_in.at[jj], p_v)
            pltpu.sync_copy(hN.at[srcv], hrows)

            @pl.loop(0, CB, unroll=4)
            def _(i):
                p16 = p_v[i, pl.ds(0, 16)]
                for t in range(8):
                    pb = jnp.full((16,), p16[t], _f32)
                    hrows[i, pl.ds(16 * t, 16)] = hrows[i, pl.ds(16 * t, 16)] * pb
            pltpu.sync_copy(hrows, agg_sh.at[dstv], add=True)
        return 0

    lax.fori_loop(0, CPS, chunk_body, 0)
    plsc.subcore_barrier()
    pltpu.sync_copy(agg_sh.at[pl.ds(s * RPW, RPW)],
                    agg_out.at[c, pl.ds(s * RPW, RPW)])


_TB = 1024


def _combine(a0, a1, d0, d1, v_ref):
    a = a0[0] + a1[0]
    den = d0[0][:, 0:8] + d1[0][:, 0:8]
    inv = (1.0 - ALPHA) / (den + 1e-16)
    inv = jnp.broadcast_to(inv[:, :, None], (_TB, 8, 16)).reshape(_TB, D)
    return a * inv + ALPHA * v_ref[...]


def _ln_elu(o, g_ref, b_ref):
    mu = jnp.mean(o, axis=-1, keepdims=True)
    dlt = o - mu
    var = jnp.mean(dlt * dlt, axis=-1, keepdims=True)
    o = dlt / jnp.sqrt(var + 1e-5) * g_ref[...] + b_ref[...]
    return jnp.where(o > 0, o, jnp.exp(o) - 1.0)


def _qkv_body(x_ref, w_ref, o_ref):
    o_ref[...] = jnp.dot(x_ref[...], w_ref[...], preferred_element_type=_f32)


def _tc_qkv(x, wcat):
    return pl.pallas_call(
        _qkv_body,
        grid=(Np // _TB,),
        in_specs=[
            pl.BlockSpec((_TB, D), lambda i: (i, 0)),
            pl.BlockSpec((D, 3 * D), lambda i: (0, 0)),
        ],
        out_specs=pl.BlockSpec((_TB, 3 * D), lambda i: (i, 0)),
        out_shape=_sds((Np, 3 * D), _f32),
    )(x, wcat)


_AGG_SPECS = [
    pl.BlockSpec((1, _TB, D), lambda i: (0, i, 0)),
    pl.BlockSpec((1, _TB, D), lambda i: (1, i, 0)),
    pl.BlockSpec((1, _TB, D), lambda i: (0, i, 0)),
    pl.BlockSpec((1, _TB, D), lambda i: (1, i, 0)),
    pl.BlockSpec((_TB, D), lambda i: (i, 0)),
]


def _fin_body(a0, a1, d0, d1, v_ref, o_ref):
    o_ref[...] = _combine(a0, a1, d0, d1, v_ref)


def _tc_fin(aggp, denp, vN):
    return pl.pallas_call(
        _fin_body,
        grid=(Np // _TB,),
        in_specs=_AGG_SPECS,
        out_specs=pl.BlockSpec((_TB, D), lambda i: (i, 0)),
        out_shape=_sds((Np, D), _f32),
    )(aggp, aggp, denp, denp, vN)


def _post_body(a0, a1, d0, d1, v_ref, x_ref, wo_ref, g_ref, b_ref, o_ref):
    h4 = _combine(a0, a1, d0, d1, v_ref)
    o = jnp.dot(h4, wo_ref[...], preferred_element_type=_f32) + x_ref[...]
    o_ref[...] = _ln_elu(o, g_ref, b_ref)


def _tc_post(aggp, denp, vN, x, wo, g, b):
    return pl.pallas_call(
        _post_body,
        grid=(Np // _TB,),
        in_specs=_AGG_SPECS + [
            pl.BlockSpec((_TB, D), lambda i: (i, 0)),
            pl.BlockSpec((D, D), lambda i: (0, 0)),
            pl.BlockSpec((1, D), lambda i: (0, 0)),
            pl.BlockSpec((1, D), lambda i: (0, 0)),
        ],
        out_specs=pl.BlockSpec((_TB, D), lambda i: (i, 0)),
        out_shape=_sds((Np, D), _f32),
    )(aggp, aggp, denp, denp, vN, x, wo, g, b)


def _final_body(a0, a1, d0, d1, v_ref, x_ref, wo_ref, g_ref, b_ref,
                wc_ref, bc_ref, o_ref):
    h4 = _combine(a0, a1, d0, d1, v_ref)
    o = jnp.dot(h4, wo_ref[...], preferred_element_type=_f32) + x_ref[...]
    o = _ln_elu(o, g_ref, b_ref)
    o_ref[...] = jnp.dot(o, wc_ref[...], preferred_element_type=_f32) + bc_ref[...]


def _tc_final(aggp, denp, vN, x, wo, g, b, wcp, bcp):
    return pl.pallas_call(
        _final_body,
        grid=(Np // _TB,),
        in_specs=_AGG_SPECS + [
            pl.BlockSpec((_TB, D), lambda i: (i, 0)),
            pl.BlockSpec((D, D), lambda i: (0, 0)),
            pl.BlockSpec((1, D), lambda i: (0, 0)),
            pl.BlockSpec((1, D), lambda i: (0, 0)),
            pl.BlockSpec((D, D), lambda i: (0, 0)),
            pl.BlockSpec((1, D), lambda i: (0, 0)),
        ],
        out_specs=pl.BlockSpec((_TB, D), lambda i: (i, 0)),
        out_shape=_sds((Np, D), _f32),
    )(aggp, aggp, denp, denp, vN, x, wo, g, b, wcp, bcp)


def _layer_sc(x, dstr, srcr, zeros128, wq, wk, wv):
    perm = jnp.asarray(_QK_PERM)
    wcat = jnp.concatenate([wq[:, perm], wk[:, perm], wv], axis=1)
    qkv = _tc_qkv(x, wcat)
    qh = qkv[:, 0:D]
    kh = qkv[:, D:2 * D]
    vN = qkv[:, 2 * D:3 * D]
    e, wmax = _sc_logits(qh, kh, dstr, srcr)
    p, denp = _sc_softmax(e, dstr, wmax, zeros128)
    h = vN
    for _ in range(HOP - 1):
        aggp = _sc_hop(h, p, srcr, dstr, zeros128)
        h = _tc_fin(aggp, denp, vN)
    aggp = _sc_hop(h, p, srcr, dstr, zeros128)
    return aggp, denp, vN


def kernel(x, edge_index, Wq1, Wk1, Wv1, Wo1, g1, b1,
           Wq2, Wk2, Wv2, Wo2, g2, b2, Wc, bc):
    src = edge_index[0]
    dst = edge_index[1]
    pad = CH * CB - E
    srcr = jnp.concatenate([src, jnp.zeros((pad,), _i32)]).reshape(CH, CB)
    dstp = jnp.concatenate([dst, jnp.zeros((pad,), _i32)])
    dstr = dstp.reshape(CH, CB)
    zeros128 = jnp.zeros((RPW, D), _f32)
    xp = jnp.pad(x, ((0, Np - N), (0, 0)))

    aggp, denp, vN = _layer_sc(xp, dstr, srcr, zeros128, Wq1, Wk1, Wv1)
    h1 = _tc_post(aggp, denp, vN, xp, Wo1, g1.reshape(1, D), b1.reshape(1, D))

    aggp, denp, vN = _layer_sc(h1, dstr, srcr, zeros128, Wq2, Wk2, Wv2)
    wcp = jnp.zeros((D, D), _f32).at[:, :C].set(Wc)
    bcp = jnp.zeros((1, D), _f32).at[0, :C].set(bc)
    logits = _tc_final(aggp, denp, vN, h1, Wo2, g2.reshape(1, D),
                       b2.reshape(1, D), wcp, bcp)
    return logits[:N, :C]


# speedup vs baseline: 1.3124x; 1.3124x over previous
import functools

import jax
import jax.numpy as jnp
import numpy as np
from jax import lax
from jax.experimental import pallas as pl
from jax.experimental.pallas import tpu as pltpu
from jax.experimental.pallas import tpu_sc as plsc

N = 10000
E = 320000
D = 128
HOP = 4
ALPHA = 0.15
C = 47

Np = 10112
CB = 128
CH_REAL = E // CB
CH = 2560
NSUB = 16
NW = 2 * NSUB
CPS = CH // NW
RPW = Np // NSUB

_mesh = plsc.VectorSubcoreMesh(core_axis_name="c", subcore_axis_name="s")
_f32 = jnp.float32
_i32 = jnp.int32


def _sds(shape, dtype):
    return jax.ShapeDtypeStruct(shape, dtype)


_QK_PERM = np.array(
    [(l * 16 + 2 * t) if l < 8 else ((15 - l) * 16 + 2 * t + 1)
     for t in range(8) for l in range(16)], dtype=np.int32)


@functools.partial(
    pl.kernel,
    out_type=(_sds((CH, CB, 16), _f32), _sds((NW, 1, 16), _f32)),
    mesh=_mesh,
    scratch_types=[
        pltpu.VMEM((CB,), _i32), pltpu.VMEM((CB,), _i32),
        pltpu.VMEM((CB,), _i32), pltpu.VMEM((CB,), _i32),
        pltpu.VMEM((CB, D), _f32), pltpu.VMEM((CB, D), _f32),
        pltpu.VMEM((CB, D), _f32), pltpu.VMEM((CB, D), _f32),
        pltpu.VMEM((CB, 16), _f32),
        pltpu.VMEM((1, 16), _f32),
        pltpu.SemaphoreType.DMA, pltpu.SemaphoreType.DMA,
        pltpu.SemaphoreType.DMA, pltpu.SemaphoreType.DMA,
    ],
)
def _sc_logits(qh, kh, dstr, srcr, e_out, wmax_out,
               idqA, idqB, idkA, idkB, qdA, qdB, ksA, ksB, e_buf, wm_v,
               semQA, semKA, semQB, semKB):
    c = lax.axis_index("c")
    s = lax.axis_index("s")
    wid = c * NSUB + s
    wm_v[0, :] = jnp.full((16,), -3e38, _f32)
    base = wid * CPS
    nreal = jnp.minimum(CH_REAL - base, CPS)

    def compute(qd, ks, jj):
        @pl.loop(0, CB)
        def _(i):
            acc = qd[i, pl.ds(0, 16)] * ks[i, pl.ds(0, 16)]
            for t in range(1, 8):
                acc = acc + qd[i, pl.ds(16 * t, 16)] * ks[i, pl.ds(16 * t, 16)]
            ef = acc + lax.rev(acc, (0,))
            ef = jnp.maximum(ef * 0.25, ef * 0.05)
            e_buf[i, pl.ds(0, 16)] = ef
            wm_v[0, :] = jnp.maximum(wm_v[0, :], ef)
        pltpu.sync_copy(e_buf, e_out.at[jj])

    pltpu.sync_copy(dstr.at[base], idqA)
    pltpu.sync_copy(srcr.at[base], idkA)
    pltpu.async_copy(qh.at[idqA], qdA, semQA)
    pltpu.async_copy(kh.at[idkA], ksA, semKA)

    def pair_body(j2, _):
        jjA = base + 2 * j2
        jjB = jjA + 1
        pltpu.sync_copy(dstr.at[jjB], idqB)
        pltpu.sync_copy(srcr.at[jjB], idkB)
        pltpu.async_copy(qh.at[idqB], qdB, semQB)
        pltpu.async_copy(kh.at[idkB], ksB, semKB)
        pltpu.make_async_copy(qh.at[idqA], qdA, semQA).wait()
        pltpu.make_async_copy(kh.at[idkA], ksA, semKA).wait()
        compute(qdA, ksA, jjA)

        @pl.when(2 * j2 + 2 < nreal)
        def _():
            pltpu.sync_copy(dstr.at[jjA + 2], idqA)
            pltpu.sync_copy(srcr.at[jjA + 2], idkA)
            pltpu.async_copy(qh.at[idqA], qdA, semQA)
            pltpu.async_copy(kh.at[idkA], ksA, semKA)

        pltpu.make_async_copy(qh.at[idqB], qdB, semQB).wait()
        pltpu.make_async_copy(kh.at[idkB], ksB, semKB).wait()
        compute(qdB, ksB, jjB)
        return 0

    lax.fori_loop(0, nreal // 2, pair_body, 0)
    pltpu.sync_copy(wm_v, wmax_out.at[wid])


@functools.partial(
    pl.kernel,
    out_type=(_sds((CH, CB, 16), _f32), _sds((2, Np, D), _f32)),
    mesh=_mesh,
    scratch_types=[
        pltpu.VMEM((CB, 16), _f32),
        pltpu.VMEM((CB, D), _f32),
        pltpu.VMEM((CB,), _i32),
        pltpu.VMEM((NW, 1, 16), _f32),
        pltpu.VMEM((1, 16), _f32),
        pltpu.VMEM_SHARED((Np, D), _f32),
    ],
)
def _sc_softmax(e_in, dstr, wmax_in, zeros128, p_out, den_out,
                eb, pwide, dstv, wm_all, gs_v, den_sh):
    c = lax.axis_index("c")
    s = lax.axis_index("s")
    wid = c * NSUB + s

    pltpu.sync_copy(zeros128, den_sh.at[pl.ds(s * RPW, RPW)])
    pltpu.sync_copy(zeros128.at[pl.ds(0, CB)], pwide)

    pltpu.sync_copy(wmax_in, wm_all)
    gs_v[0, :] = jnp.full((16,), -3e38, _f32)

    def mx(t, _):
        gs_v[0, :] = jnp.maximum(gs_v[0, :], wm_all[t, 0, :])
        return 0
    lax.fori_loop(0, NW, mx, 0)
    gs = gs_v[0, :]
    plsc.subcore_barrier()

    def chunk_body(j, _):
        jj = wid * CPS + j

        @pl.when(jj < CH_REAL)
        def _():
            pltpu.sync_copy(e_in.at[jj], eb)
            pltpu.sync_copy(dstr.at[jj], dstv)

            @pl.loop(0, CB)
            def _(i):
                p = jnp.exp(eb[i, pl.ds(0, 16)] - gs)
                eb[i, pl.ds(0, 16)] = p
                pwide[i, pl.ds(0, 16)] = p
            pltpu.sync_copy(eb, p_out.at[jj])
            pltpu.sync_copy(pwide, den_sh.at[dstv], add=True)
        return 0

    lax.fori_loop(0, CPS, chunk_body, 0)
    plsc.subcore_barrier()
    pltpu.sync_copy(den_sh.at[pl.ds(s * RPW, RPW)],
                    den_out.at[c, pl.ds(s * RPW, RPW)])


@functools.partial(
    pl.kernel,
    out_type=_sds((2, Np, D), _f32),
    mesh=_mesh,
    scratch_types=[
        pltpu.VMEM((CB,), _i32), pltpu.VMEM((CB,), _i32),
        pltpu.VMEM((CB,), _i32),
        pltpu.VMEM((CB, 16), _f32),
        pltpu.VMEM((CB, D), _f32), pltpu.VMEM((CB, D), _f32),
        pltpu.VMEM_SHARED((Np, D), _f32),
        pltpu.SemaphoreType.DMA, pltpu.SemaphoreType.DMA,
    ],
)
def _sc_hop(hN, p_in, srcr, dstr, zeros128, agg_out,
            srcA, srcB, dstv, p_v, hrA, hrB, agg_sh, semA, semB):
    c = lax.axis_index("c")
    s = lax.axis_index("s")
    wid = c * NSUB + s

    pltpu.sync_copy(zeros128, agg_sh.at[pl.ds(s * RPW, RPW)])
    plsc.subcore_barrier()
    base = wid * CPS
    nreal = jnp.minimum(CH_REAL - base, CPS)

    def compute(hr, jj):
        pltpu.sync_copy(p_in.at[jj], p_v)

        @pl.loop(0, CB)
        def _(i):
            p16 = p_v[i, pl.ds(0, 16)]
            for t in range(8):
                pb = jnp.full((16,), p16[t], _f32)
                hr[i, pl.ds(16 * t, 16)] = hr[i, pl.ds(16 * t, 16)] * pb
        pltpu.sync_copy(dstr.at[jj], dstv)
        pltpu.sync_copy(hr, agg_sh.at[dstv], add=True)

    pltpu.sync_copy(srcr.at[base], srcA)
    pltpu.async_copy(hN.at[srcA], hrA, semA)

    def pair_body(j2, _):
        jjA = base + 2 * j2
        jjB = jjA + 1
        pltpu.sync_copy(srcr.at[jjB], srcB)
        pltpu.async_copy(hN.at[srcB], hrB, semB)
        pltpu.make_async_copy(hN.at[srcA], hrA, semA).wait()
        compute(hrA, jjA)

        @pl.when(2 * j2 + 2 < nreal)
        def _():
            pltpu.sync_copy(srcr.at[jjA + 2], srcA)
            pltpu.async_copy(hN.at[srcA], hrA, semA)

        pltpu.make_async_copy(hN.at[srcB], hrB, semB).wait()
        compute(hrB, jjB)
        return 0

    lax.fori_loop(0, nreal // 2, pair_body, 0)
    plsc.subcore_barrier()
    pltpu.sync_copy(agg_sh.at[pl.ds(s * RPW, RPW)],
                    agg_out.at[c, pl.ds(s * RPW, RPW)])


_TB = 632


def _combine(a0, a1, d0, d1, v_ref):
    a = a0[0] + a1[0]
    den = d0[0][:, 0:8] + d1[0][:, 0:8]
    inv = (1.0 - ALPHA) / (den + 1e-16)
    inv = jnp.broadcast_to(inv[:, :, None], (_TB, 8, 16)).reshape(_TB, D)
    return a * inv + ALPHA * v_ref[...]


def _ln_elu(o, g_ref, b_ref):
    mu = jnp.mean(o, axis=-1, keepdims=True)
    dlt = o - mu
    var = jnp.mean(dlt * dlt, axis=-1, keepdims=True)
    o = dlt / jnp.sqrt(var + 1e-5) * g_ref[...] + b_ref[...]
    return jnp.where(o > 0, o, jnp.exp(o) - 1.0)


def _qkv_body(x_ref, w_ref, o_ref):
    o_ref[...] = jnp.dot(x_ref[...], w_ref[...], preferred_element_type=_f32)


def _tc_qkv(x, wcat):
    return pl.pallas_call(
        _qkv_body,
        grid=(Np // _TB,),
        in_specs=[
            pl.BlockSpec((_TB, D), lambda i: (i, 0)),
            pl.BlockSpec((D, 3 * D), lambda i: (0, 0)),
        ],
        out_specs=pl.BlockSpec((_TB, 3 * D), lambda i: (i, 0)),
        out_shape=_sds((Np, 3 * D), _f32),
    )(x, wcat)


_AGG_SPECS = [
    pl.BlockSpec((1, _TB, D), lambda i: (0, i, 0)),
    pl.BlockSpec((1, _TB, D), lambda i: (1, i, 0)),
    pl.BlockSpec((1, _TB, D), lambda i: (0, i, 0)),
    pl.BlockSpec((1, _TB, D), lambda i: (1, i, 0)),
    pl.BlockSpec((_TB, D), lambda i: (i, 0)),
]


def _fin_body(a0, a1, d0, d1, v_ref, o_ref):
    o_ref[...] = _combine(a0, a1, d0, d1, v_ref)


def _tc_fin(aggp, denp, vN):
    return pl.pallas_call(
        _fin_body,
        grid=(Np // _TB,),
        in_specs=_AGG_SPECS,
        out_specs=pl.BlockSpec((_TB, D), lambda i: (i, 0)),
        out_shape=_sds((Np, D), _f32),
    )(aggp, aggp, denp, denp, vN)


def _post_body(a0, a1, d0, d1, v_ref, x_ref, wo_ref, g_ref, b_ref, o_ref):
    h4 = _combine(a0, a1, d0, d1, v_ref)
    o = jnp.dot(h4, wo_ref[...], preferred_element_type=_f32) + x_ref[...]
    o_ref[...] = _ln_elu(o, g_ref, b_ref)


def _tc_post(aggp, denp, vN, x, wo, g, b):
    return pl.pallas_call(
        _post_body,
        grid=(Np // _TB,),
        in_specs=_AGG_SPECS + [
            pl.BlockSpec((_TB, D), lambda i: (i, 0)),
            pl.BlockSpec((D, D), lambda i: (0, 0)),
            pl.BlockSpec((1, D), lambda i: (0, 0)),
            pl.BlockSpec((1, D), lambda i: (0, 0)),
        ],
        out_specs=pl.BlockSpec((_TB, D), lambda i: (i, 0)),
        out_shape=_sds((Np, D), _f32),
    )(aggp, aggp, denp, denp, vN, x, wo, g, b)


def _final_body(a0, a1, d0, d1, v_ref, x_ref, wo_ref, g_ref, b_ref,
                wc_ref, bc_ref, o_ref):
    h4 = _combine(a0, a1, d0, d1, v_ref)
    o = jnp.dot(h4, wo_ref[...], preferred_element_type=_f32) + x_ref[...]
    o = _ln_elu(o, g_ref, b_ref)
    o_ref[...] = jnp.dot(o, wc_ref[...], preferred_element_type=_f32) + bc_ref[...]


def _tc_final(aggp, denp, vN, x, wo, g, b, wcp, bcp):
    return pl.pallas_call(
        _final_body,
        grid=(Np // _TB,),
        in_specs=_AGG_SPECS + [
            pl.BlockSpec((_TB, D), lambda i: (i, 0)),
            pl.BlockSpec((D, D), lambda i: (0, 0)),
            pl.BlockSpec((1, D), lambda i: (0, 0)),
            pl.BlockSpec((1, D), lambda i: (0, 0)),
            pl.BlockSpec((D, D), lambda i: (0, 0)),
            pl.BlockSpec((1, D), lambda i: (0, 0)),
        ],
        out_specs=pl.BlockSpec((_TB, D), lambda i: (i, 0)),
        out_shape=_sds((Np, D), _f32),
    )(aggp, aggp, denp, denp, vN, x, wo, g, b, wcp, bcp)


def _layer_sc(x, dstr, srcr, zeros128, wq, wk, wv):
    perm = jnp.asarray(_QK_PERM)
    wcat = jnp.concatenate([wq[:, perm], wk[:, perm], wv], axis=1)
    qkv = _tc_qkv(x, wcat)
    qh = qkv[:, 0:D]
    kh = qkv[:, D:2 * D]
    vN = qkv[:, 2 * D:3 * D]
    e, wmax = _sc_logits(qh, kh, dstr, srcr)
    p, denp = _sc_softmax(e, dstr, wmax, zeros128)
    h = vN
    for _ in range(HOP - 1):
        aggp = _sc_hop(h, p, srcr, dstr, zeros128)
        h = _tc_fin(aggp, denp, vN)
    aggp = _sc_hop(h, p, srcr, dstr, zeros128)
    return aggp, denp, vN


def kernel(x, edge_index, Wq1, Wk1, Wv1, Wo1, g1, b1,
           Wq2, Wk2, Wv2, Wo2, g2, b2, Wc, bc):
    src = edge_index[0]
    dst = edge_index[1]
    pad = CH * CB - E
    srcr = jnp.concatenate([src, jnp.zeros((pad,), _i32)]).reshape(CH, CB)
    dstp = jnp.concatenate([dst, jnp.zeros((pad,), _i32)])
    dstr = dstp.reshape(CH, CB)
    zeros128 = jnp.zeros((RPW, D), _f32)
    xp = jnp.pad(x, ((0, Np - N), (0, 0)))

    aggp, denp, vN = _layer_sc(xp, dstr, srcr, zeros128, Wq1, Wk1, Wv1)
    h1 = _tc_post(aggp, denp, vN, xp, Wo1, g1.reshape(1, D), b1.reshape(1, D))

    aggp, denp, vN = _layer_sc(h1, dstr, srcr, zeros128, Wq2, Wk2, Wv2)
    wcp = jnp.zeros((D, D), _f32).at[:, :C].set(Wc)
    bcp = jnp.zeros((1, D), _f32).at[0, :C].set(bc)
    logits = _tc_final(aggp, denp, vN, h1, Wo2, g2.reshape(1, D),
                       b2.reshape(1, D), wcp, bcp)
    return logits[:N, :C]


# speedup vs baseline: 1.3130x; 1.0005x over previous
import functools

import jax
import jax.numpy as jnp
import numpy as np
from jax import lax
from jax.experimental import pallas as pl
from jax.experimental.pallas import tpu as pltpu
from jax.experimental.pallas import tpu_sc as plsc

N = 10000
E = 320000
D = 128
HOP = 4
ALPHA = 0.15
C = 47

Np = 10112
CB = 128
CH_REAL = E // CB
CH = 2560
NSUB = 16
NW = 2 * NSUB
CPS = CH // NW
RPW = Np // NSUB

_mesh = plsc.VectorSubcoreMesh(core_axis_name="c", subcore_axis_name="s")
_f32 = jnp.float32
_i32 = jnp.int32


def _sds(shape, dtype):
    return jax.ShapeDtypeStruct(shape, dtype)


_QK_PERM = np.array(
    [(l * 16 + 2 * t) if l < 8 else ((15 - l) * 16 + 2 * t + 1)
     for t in range(8) for l in range(16)], dtype=np.int32)


@functools.partial(
    pl.kernel,
    out_type=(_sds((CH, CB, 16), _f32), _sds((NW, 1, 16), _f32)),
    mesh=_mesh,
    scratch_types=[
        pltpu.VMEM((CB,), _i32), pltpu.VMEM((CB,), _i32),
        pltpu.VMEM((CB,), _i32), pltpu.VMEM((CB,), _i32),
        pltpu.VMEM((CB, D), _f32), pltpu.VMEM((CB, D), _f32),
        pltpu.VMEM((CB, D), _f32), pltpu.VMEM((CB, D), _f32),
        pltpu.VMEM((CB, 16), _f32),
        pltpu.VMEM((1, 16), _f32),
        pltpu.SemaphoreType.DMA, pltpu.SemaphoreType.DMA,
        pltpu.SemaphoreType.DMA, pltpu.SemaphoreType.DMA,
    ],
)
def _sc_logits(qh, kh, dstr, srcr, e_out, wmax_out,
               idqA, idqB, idkA, idkB, qdA, qdB, ksA, ksB, e_buf, wm_v,
               semQA, semKA, semQB, semKB):
    c = lax.axis_index("c")
    s = lax.axis_index("s")
    wid = c * NSUB + s
    wm_v[0, :] = jnp.full((16,), -3e38, _f32)
    base = wid * CPS
    nreal = jnp.minimum(CH_REAL - base, CPS)

    def compute(qd, ks, jj):
        @pl.loop(0, CB)
        def _(i):
            acc = qd[i, pl.ds(0, 16)] * ks[i, pl.ds(0, 16)]
            for t in range(1, 8):
                acc = acc + qd[i, pl.ds(16 * t, 16)] * ks[i, pl.ds(16 * t, 16)]
            ef = acc + lax.rev(acc, (0,))
            ef = jnp.maximum(ef * 0.25, ef * 0.05)
            e_buf[i, pl.ds(0, 16)] = ef
            wm_v[0, :] = jnp.maximum(wm_v[0, :], ef)
        pltpu.sync_copy(e_buf, e_out.at[jj])

    pltpu.sync_copy(dstr.at[base], idqA)
    pltpu.sync_copy(srcr.at[base], idkA)
    pltpu.async_copy(qh.at[idqA], qdA, semQA)
    pltpu.async_copy(kh.at[idkA], ksA, semKA)

    def pair_body(j2, _):
        jjA = base + 2 * j2
        jjB = jjA + 1
        pltpu.sync_copy(dstr.at[jjB], idqB)
        pltpu.sync_copy(srcr.at[jjB], idkB)
        pltpu.async_copy(qh.at[idqB], qdB, semQB)
        pltpu.async_copy(kh.at[idkB], ksB, semKB)
        pltpu.make_async_copy(qh.at[idqA], qdA, semQA).wait()
        pltpu.make_async_copy(kh.at[idkA], ksA, semKA).wait()
        compute(qdA, ksA, jjA)

        @pl.when(2 * j2 + 2 < nreal)
        def _():
            pltpu.sync_copy(dstr.at[jjA + 2], idqA)
            pltpu.sync_copy(srcr.at[jjA + 2], idkA)
            pltpu.async_copy(qh.at[idqA], qdA, semQA)
            pltpu.async_copy(kh.at[idkA], ksA, semKA)

        pltpu.make_async_copy(qh.at[idqB], qdB, semQB).wait()
        pltpu.make_async_copy(kh.at[idkB], ksB, semKB).wait()
        compute(qdB, ksB, jjB)
        return 0

    lax.fori_loop(0, nreal // 2, pair_body, 0)
    pltpu.sync_copy(wm_v, wmax_out.at[wid])


@functools.partial(
    pl.kernel,
    out_type=(_sds((CH, CB, 16), _f32), _sds((2, Np, D), _f32)),
    mesh=_mesh,
    scratch_types=[
        pltpu.VMEM((CB, 16), _f32),
        pltpu.VMEM((CB, D), _f32),
        pltpu.VMEM((CB,), _i32),
        pltpu.VMEM((NW, 1, 16), _f32),
        pltpu.VMEM((1, 16), _f32),
        pltpu.VMEM_SHARED((Np, D), _f32),
    ],
)
def _sc_softmax(e_in, dstr, wmax_in, zeros128, p_out, den_out,
                eb, pwide, dstv, wm_all, gs_v, den_sh):
    c = lax.axis_index("c")
    s = lax.axis_index("s")
    wid = c * NSUB + s

    pltpu.sync_copy(zeros128, den_sh.at[pl.ds(s * RPW, RPW)])
    pltpu.sync_copy(zeros128.at[pl.ds(0, CB)], pwide)

    pltpu.sync_copy(wmax_in, wm_all)
    gs_v[0, :] = jnp.full((16,), -3e38, _f32)

    def mx(t, _):
        gs_v[0, :] = jnp.maximum(gs_v[0, :], wm_all[t, 0, :])
        return 0
    lax.fori_loop(0, NW, mx, 0)
    gs = gs_v[0, :]
    plsc.subcore_barrier()
    base = wid * CPS
    nreal = jnp.minimum(CH_REAL - base, CPS)

    def chunk_body(j, _):
        jj = base + j
        pltpu.sync_copy(e_in.at[jj], eb)
        pltpu.sync_copy(dstr.at[jj], dstv)

        @pl.loop(0, CB)
        def _(i):
            p = jnp.exp(eb[i, pl.ds(0, 16)] - gs)
            eb[i, pl.ds(0, 16)] = p
            pwide[i, pl.ds(0, 16)] = p
        pltpu.sync_copy(eb, p_out.at[jj])
        pltpu.sync_copy(pwide, den_sh.at[dstv], add=True)
        return 0

    lax.fori_loop(0, nreal, chunk_body, 0)
    plsc.subcore_barrier()
    pltpu.sync_copy(den_sh.at[pl.ds(s * RPW, RPW)],
                    den_out.at[c, pl.ds(s * RPW, RPW)])


@functools.partial(
    pl.kernel,
    out_type=_sds((2, Np, D), _f32),
    mesh=_mesh,
    scratch_types=[
        pltpu.VMEM((CB,), _i32), pltpu.VMEM((CB,), _i32),
        pltpu.VMEM((CB,), _i32), pltpu.VMEM((CB,), _i32),
        pltpu.VMEM((CB, 16), _f32),
        pltpu.VMEM((CB, D), _f32), pltpu.VMEM((CB, D), _f32),
        pltpu.VMEM_SHARED((Np, D), _f32),
        pltpu.SemaphoreType.DMA, pltpu.SemaphoreType.DMA,
        pltpu.SemaphoreType.DMA,
    ],
)
def _sc_hop(hN, p_in, srcr, dstr, zeros128, agg_out,
            srcA, srcB, dstA, dstB, p_v, hrA, hrB, agg_sh, semA, semB, semSB):
    c = lax.axis_index("c")
    s = lax.axis_index("s")
    wid = c * NSUB + s

    pltpu.sync_copy(zeros128, agg_sh.at[pl.ds(s * RPW, RPW)])
    plsc.subcore_barrier()
    base = wid * CPS
    nreal = jnp.minimum(CH_REAL - base, CPS)

    def compute(hr, dstv, jj):
        pltpu.sync_copy(p_in.at[jj], p_v)

        @pl.loop(0, CB)
        def _(i):
            p16 = p_v[i, pl.ds(0, 16)]
            for t in range(8):
                pb = jnp.full((16,), p16[t], _f32)
                hr[i, pl.ds(16 * t, 16)] = hr[i, pl.ds(16 * t, 16)] * pb
        pltpu.sync_copy(dstr.at[jj], dstv)

    pltpu.sync_copy(srcr.at[base], srcA)
    pltpu.async_copy(hN.at[srcA], hrA, semA)

    def pair_body(j2, _):
        jjA = base + 2 * j2
        jjB = jjA + 1

        @pl.when(j2 > 0)
        def _():
            pltpu.make_async_copy(hrB, agg_sh.at[dstB], semSB).wait()
        pltpu.sync_copy(srcr.at[jjB], srcB)
        pltpu.async_copy(hN.at[srcB], hrB, semB)
        pltpu.make_async_copy(hN.at[srcA], hrA, semA).wait()
        compute(hrA, dstA, jjA)
        pltpu.sync_copy(hrA, agg_sh.at[dstA], add=True)

        @pl.when(2 * j2 + 2 < nreal)
        def _():
            pltpu.sync_copy(srcr.at[jjA + 2], srcA)
            pltpu.async_copy(hN.at[srcA], hrA, semA)

        pltpu.make_async_copy(hN.at[srcB], hrB, semB).wait()
        compute(hrB, dstB, jjB)
        pltpu.async_copy(hrB, agg_sh.at[dstB], semSB, add=True)
        return 0

    lax.fori_loop(0, nreal // 2, pair_body, 0)
    pltpu.make_async_copy(hrB, agg_sh.at[dstB], semSB).wait()
    plsc.subcore_barrier()
    pltpu.sync_copy(agg_sh.at[pl.ds(s * RPW, RPW)],
                    agg_out.at[c, pl.ds(s * RPW, RPW)])


_TB = 632


def _combine(a0, a1, d0, d1, v_ref):
    a = a0[0] + a1[0]
    den = d0[0][:, 0:8] + d1[0][:, 0:8]
    inv = (1.0 - ALPHA) / (den + 1e-16)
    inv = jnp.broadcast_to(inv[:, :, None], (_TB, 8, 16)).reshape(_TB, D)
    return a * inv + ALPHA * v_ref[...]


def _ln_elu(o, g_ref, b_ref):
    mu = jnp.mean(o, axis=-1, keepdims=True)
    dlt = o - mu
    var = jnp.mean(dlt * dlt, axis=-1, keepdims=True)
    o = dlt / jnp.sqrt(var + 1e-5) * g_ref[...] + b_ref[...]
    return jnp.where(o > 0, o, jnp.exp(o) - 1.0)


def _qkv_body(x_ref, w_ref, o_ref):
    o_ref[...] = jnp.dot(x_ref[...], w_ref[...], preferred_element_type=_f32)


def _tc_qkv(x, wcat):
    return pl.pallas_call(
        _qkv_body,
        grid=(Np // _TB,),
        in_specs=[
            pl.BlockSpec((_TB, D), lambda i: (i, 0)),
            pl.BlockSpec((D, 3 * D), lambda i: (0, 0)),
        ],
        out_specs=pl.BlockSpec((_TB, 3 * D), lambda i: (i, 0)),
        out_shape=_sds((Np, 3 * D), _f32),
    )(x, wcat)


_AGG_SPECS = [
    pl.BlockSpec((1, _TB, D), lambda i: (0, i, 0)),
    pl.BlockSpec((1, _TB, D), lambda i: (1, i, 0)),
    pl.BlockSpec((1, _TB, D), lambda i: (0, i, 0)),
    pl.BlockSpec((1, _TB, D), lambda i: (1, i, 0)),
    pl.BlockSpec((_TB, D), lambda i: (i, 0)),
]


def _fin_body(a0, a1, d0, d1, v_ref, o_ref):
    o_ref[...] = _combine(a0, a1, d0, d1, v_ref)


def _tc_fin(aggp, denp, vN):
    return pl.pallas_call(
        _fin_body,
        grid=(Np // _TB,),
        in_specs=_AGG_SPECS,
        out_specs=pl.BlockSpec((_TB, D), lambda i: (i, 0)),
        out_shape=_sds((Np, D), _f32),
    )(aggp, aggp, denp, denp, vN)


def _post_body(a0, a1, d0, d1, v_ref, x_ref, wo_ref, g_ref, b_ref, o_ref):
    h4 = _combine(a0, a1, d0, d1, v_ref)
    o = jnp.dot(h4, wo_ref[...], preferred_element_type=_f32) + x_ref[...]
    o_ref[...] = _ln_elu(o, g_ref, b_ref)


def _tc_post(aggp, denp, vN, x, wo, g, b):
    return pl.pallas_call(
        _post_body,
        grid=(Np // _TB,),
        in_specs=_AGG_SPECS + [
            pl.BlockSpec((_TB, D), lambda i: (i, 0)),
            pl.BlockSpec((D, D), lambda i: (0, 0)),
            pl.BlockSpec((1, D), lambda i: (0, 0)),
            pl.BlockSpec((1, D), lambda i: (0, 0)),
        ],
        out_specs=pl.BlockSpec((_TB, D), lambda i: (i, 0)),
        out_shape=_sds((Np, D), _f32),
    )(aggp, aggp, denp, denp, vN, x, wo, g, b)


def _final_body(a0, a1, d0, d1, v_ref, x_ref, wo_ref, g_ref, b_ref,
                wc_ref, bc_ref, o_ref):
    h4 = _combine(a0, a1, d0, d1, v_ref)
    o = jnp.dot(h4, wo_ref[...], preferred_element_type=_f32) + x_ref[...]
    o = _ln_elu(o, g_ref, b_ref)
    o_ref[...] = jnp.dot(o, wc_ref[...], preferred_element_type=_f32) + bc_ref[...]


def _tc_final(aggp, denp, vN, x, wo, g, b, wcp, bcp):
    return pl.pallas_call(
        _final_body,
        grid=(Np // _TB,),
        in_specs=_AGG_SPECS + [
            pl.BlockSpec((_TB, D), lambda i: (i, 0)),
            pl.BlockSpec((D, D), lambda i: (0, 0)),
            pl.BlockSpec((1, D), lambda i: (0, 0)),
            pl.BlockSpec((1, D), lambda i: (0, 0)),
            pl.BlockSpec((D, D), lambda i: (0, 0)),
            pl.BlockSpec((1, D), lambda i: (0, 0)),
        ],
        out_specs=pl.BlockSpec((_TB, D), lambda i: (i, 0)),
        out_shape=_sds((Np, D), _f32),
    )(aggp, aggp, denp, denp, vN, x, wo, g, b, wcp, bcp)


def _layer_sc(x, dstr, srcr, zeros128, wq, wk, wv):
    perm = jnp.asarray(_QK_PERM)
    wcat = jnp.concatenate([wq[:, perm], wk[:, perm], wv], axis=1)
    qkv = _tc_qkv(x, wcat)
    qh = qkv[:, 0:D]
    kh = qkv[:, D:2 * D]
    vN = qkv[:, 2 * D:3 * D]
    e, wmax = _sc_logits(qh, kh, dstr, srcr)
    p, denp = _sc_softmax(e, dstr, wmax, zeros128)
    h = vN
    for _ in range(HOP - 1):
        aggp = _sc_hop(h, p, srcr, dstr, zeros128)
        h = _tc_fin(aggp, denp, vN)
    aggp = _sc_hop(h, p, srcr, dstr, zeros128)
    return aggp, denp, vN


def kernel(x, edge_index, Wq1, Wk1, Wv1, Wo1, g1, b1,
           Wq2, Wk2, Wv2, Wo2, g2, b2, Wc, bc):
    src = edge_index[0]
    dst = edge_index[1]
    pad = CH * CB - E
    srcr = jnp.concatenate([src, jnp.zeros((pad,), _i32)]).reshape(CH, CB)
    dstp = jnp.concatenate([dst, jnp.zeros((pad,), _i32)])
    dstr = dstp.reshape(CH, CB)
    zeros128 = jnp.zeros((RPW, D), _f32)
    xp = jnp.pad(x, ((0, Np - N), (0, 0)))

    aggp, denp, vN = _layer_sc(xp, dstr, srcr, zeros128, Wq1, Wk1, Wv1)
    h1 = _tc_post(aggp, denp, vN, xp, Wo1, g1.reshape(1, D), b1.reshape(1, D))

    aggp, denp, vN = _layer_sc(h1, dstr, srcr, zeros128, Wq2, Wk2, Wv2)
    wcp = jnp.zeros((D, D), _f32).at[:, :C].set(Wc)
    bcp = jnp.zeros((1, D), _f32).at[0, :C].set(bc)
    logits = _tc_final(aggp, denp, vN, h1, Wo2, g2.reshape(1, D),
                       b2.reshape(1, D), wcp, bcp)
    return logits[:N, :C]
